# baseline (speedup 1.0000x reference)
import functools

import jax
import jax.numpy as jnp
from jax import lax
from jax.experimental import pallas as pl
from jax.experimental.pallas import tpu as pltpu
from jax.experimental.pallas import tpu_sc as plsc

N_TILES = 64
HGRID = 8
CIN = 768
COUT = 768
B_TOKENS = 8192
A_SCALE = 16.0
A_BIAS = 0.5

R = 256
NBLK = 96
PB = NBLK * R
CHALF = COUT // 2
DCH = 128

NC = 2
NS = 16
NW = NC * NS
TOK_W = B_TOKENS // NW
NVEC = TOK_W // 16

_MESH = plsc.VectorSubcoreMesh(core_axis_name="c", subcore_axis_name="s",
                               num_cores=NC, num_subcores=NS)
_SC_PARAMS = pltpu.CompilerParams(needs_layout_passes=False)


def _worker_id():
    return lax.axis_index("s") * NC + lax.axis_index("c")


def _floor_i32(v):
    t = v.astype(jnp.int32)
    return jnp.where(t.astype(jnp.float32) > v, t - 1, t)


@functools.partial(
    pl.kernel,
    out_type=(
        jax.ShapeDtypeStruct((B_TOKENS,), jnp.int32),
        jax.ShapeDtypeStruct((B_TOKENS,), jnp.int32),
        jax.ShapeDtypeStruct((NW, N_TILES), jnp.int32),
    ),
    mesh=_MESH,
    compiler_params=_SC_PARAMS,
    scratch_types=(
        pltpu.VMEM((TOK_W, 2), jnp.float32),
        pltpu.VMEM((TOK_W,), jnp.int32),
        pltpu.VMEM((TOK_W,), jnp.int32),
        pltpu.VMEM((N_TILES,), jnp.int32),
    ),
)
def _route(coords, tiles_h, ranks_h, lcounts_h, cc_v, tl_v, rk_v, cnt_v):
    w = _worker_id()
    base = w * TOK_W
    pltpu.sync_copy(coords.at[pl.ds(base, TOK_W)], cc_v)
    for c in range(N_TILES // 16):
        cnt_v[pl.ds(c * 16, 16)] = jnp.zeros((16,), jnp.int32)

    lane = lax.iota(jnp.int32, 16)
    zeros16 = jnp.zeros((16,), jnp.int32)

    def body(k, _):
        sl = pl.ds(k * 16, 16)
        tok = lane + k * 16
        cx = plsc.load_gather(cc_v, [tok, zeros16])
        cy = plsc.load_gather(cc_v, [tok, zeros16 + 1])
        mx = _floor_i32(cx * A_SCALE + A_BIAS) & (HGRID - 1)
        my = _floor_i32(cy * A_SCALE + A_BIAS) & (HGRID - 1)
        tile = mx * HGRID + my
        old = plsc.load_gather(cnt_v, [tile])
        rank = jnp.zeros((16,), jnp.int32)
        total = jnp.zeros((16,), jnp.int32)
        for l in range(16):
            tl = jnp.sum(jnp.where(lane == l, tile, 0))
            eq = tile == tl
            rank = rank + jnp.where(eq & (lane > l), 1, 0)
            total = total + jnp.where(eq, 1, 0)
        tl_v[sl] = tile
        rk_v[sl] = old + rank
        plsc.store_scatter(cnt_v, [tile], old + total)
        return 0

    lax.fori_loop(0, NVEC, body, 0)
    pltpu.sync_copy(tl_v, tiles_h.at[pl.ds(base, TOK_W)])
    pltpu.sync_copy(rk_v, ranks_h.at[pl.ds(base, TOK_W)])
    pltpu.sync_copy(cnt_v, lcounts_h.at[w])


@functools.partial(
    pl.kernel,
    out_type=(
        jax.ShapeDtypeStruct((PB, CIN), jnp.float32),
        jax.ShapeDtypeStruct((NW * 2, DCH), jnp.int32),
        jax.ShapeDtypeStruct((3, NBLK), jnp.int32),
    ),
    mesh=_MESH,
    compiler_params=_SC_PARAMS,
    scratch_types=(
        pltpu.VMEM((NW, N_TILES), jnp.int32),
        pltpu.VMEM((N_TILES,), jnp.int32),
        pltpu.VMEM((TOK_W,), jnp.int32),
        pltpu.VMEM((TOK_W,), jnp.int32),
        pltpu.VMEM((2, DCH), jnp.int32),
        pltpu.VMEM((NBLK,), jnp.int32),
        pltpu.VMEM((3, NBLK), jnp.int32),
        pltpu.VMEM((DCH, CIN), jnp.float32),
        pltpu.SemaphoreType.DMA,
    ),
)
def _dispatch(in_feats, tiles_h, ranks_h, lcounts_h,
              xpad_h, pos_h, meta_h,
              lc_v, base_v, tl_v, rk_v, pos_v, tm_v, meta_v, xb_v, sem):
    w = _worker_id()
    base = w * TOK_W
    pltpu.sync_copy(lcounts_h, lc_v)
    pltpu.sync_copy(tiles_h.at[pl.ds(base, TOK_W)], tl_v)
    pltpu.sync_copy(ranks_h.at[pl.ds(base, TOK_W)], rk_v)

    lane = lax.iota(jnp.int32, 16)
    for c in range(NBLK // 16):
        tm_v[pl.ds(c * 16, 16)] = jnp.zeros((16,), jnp.int32)
    used = jnp.int32(0)
    carry = jnp.int32(0)
    for c in range(N_TILES // 16):
        sl = pl.ds(c * 16, 16)

        def red(wp, acc):
            tot, pre = acc
            v = lc_v[wp, sl]
            tot = tot + v
            pre = pre + jnp.where(wp < w, v, 0)
            return (tot, pre)

        tot, pre = lax.fori_loop(
            0, NW, red, (jnp.zeros((16,), jnp.int32), jnp.zeros((16,), jnp.int32)))
        nb = (tot + (R - 1)) // R
        bstart = jnp.cumsum(nb) - nb + carry
        carry = carry + jnp.sum(nb)
        base_v[sl] = bstart * R + pre
        tvec = lane + c * 16
        plsc.store_scatter(tm_v, [jnp.minimum(bstart, NBLK - 1)],
                           tvec, mask=nb > 0)
        used = used + jnp.sum(nb)

    for k in range(NVEC):
        sl = pl.ds((k % (NVEC // 2)) * 16, 16)
        t = tl_v[pl.ds(k * 16, 16)]
        p = plsc.load_gather(base_v, [t]) + rk_v[pl.ds(k * 16, 16)]
        pos_v[k // (NVEC // 2), sl] = p

    for c in range(2):
        pltpu.sync_copy(in_feats.at[pl.ds(base + c * DCH, DCH)], xb_v)
        pltpu.async_copy(xb_v, xpad_h.at[pos_v.at[c]], sem).wait()
        pltpu.sync_copy(pos_v.at[c], pos_h.at[w * 2 + c])

    @pl.when(w == 0)
    def _():
        cmax = jnp.int32(0)
        for c in range(NBLK // 16):
            sl = pl.ds(c * 16, 16)
            v = jnp.maximum(plsc.cummax(tm_v[sl]), cmax)
            meta_v[0, sl] = v
            tm_v[sl] = v
            cmax = jnp.max(v)
            blk = lane + c * 16
            meta_v[1, sl] = jnp.where(blk < used, blk, NBLK - 1)
        fcarry = jnp.int32(0)
        for c in range(NBLK // 16):
            sl = pl.ds(c * 16, 16)
            blk = lane + c * 16
            cur = tm_v[sl]
            prev = plsc.load_gather(tm_v, [jnp.maximum(blk - 1, 0)])
            diff = jnp.where((cur != prev) & (blk > 0), 1, 0)
            fid = jnp.cumsum(diff) + fcarry
            meta_v[2, sl] = fid
            fcarry = jnp.max(fid)
        pltpu.sync_copy(meta_v, meta_h)


NSLOT = 4


def _matmul_body(meta_ref, x_ref, w_hbm, b_ref, o_ref, wbuf, sems):
    i = pl.program_id(0)
    fid = meta_ref[2, i]
    slot = lax.rem(fid, NSLOT)

    def start_fetch(step_fid, step_tile):
        s = lax.rem(step_fid, NSLOT)
        pltpu.make_async_copy(w_hbm.at[step_tile], wbuf.at[s], sems.at[s]).start()

    @pl.when(i == 0)
    def _():
        start_fetch(meta_ref[2, 0], meta_ref[0, 0])
        f1, f2, f3 = meta_ref[2, 1], meta_ref[2, 2], meta_ref[2, 3]

        @pl.when(f1 > 0)
        def _():
            start_fetch(f1, meta_ref[0, 1])

        @pl.when(f2 > f1)
        def _():
            start_fetch(f2, meta_ref[0, 2])

        @pl.when(f3 > f2)
        def _():
            start_fetch(f3, meta_ref[0, 3])

    @pl.when((i > 0) & (i + 3 < NBLK))
    def _():
        fa, fb = meta_ref[2, i + 3], meta_ref[2, i + 2]

        @pl.when(fa > fb)
        def _():
            start_fetch(fa, meta_ref[0, i + 3])

    prev_fid = meta_ref[2, jnp.maximum(i - 1, 0)]

    @pl.when((i == 0) | (fid > prev_fid))
    def _():
        pltpu.make_async_copy(
            w_hbm.at[meta_ref[0, i]], wbuf.at[slot], sems.at[slot]).wait()

    x = x_ref[...]
    dn = (((1,), (1,)), ((), ()))
    acc = lax.dot_general(x, wbuf[slot], dn, preferred_element_type=jnp.float32)
    acc = acc + b_ref[...]
    o_ref[...] = jnp.where(acc >= 0, acc, 0.2 * acc)


def _grouped_matmul(meta, x_padded, W, bias2d):
    grid_spec = pltpu.PrefetchScalarGridSpec(
        num_scalar_prefetch=1,
        grid=(NBLK,),
        in_specs=[
            pl.BlockSpec((R, CIN), lambda i, m: (m[1, i], 0)),
            pl.BlockSpec(memory_space=pl.ANY),
            pl.BlockSpec((1, COUT), lambda i, m: (0, 0)),
        ],
        out_specs=pl.BlockSpec((R, COUT), lambda i, m: (m[1, i], 0)),
        scratch_shapes=[
            pltpu.VMEM((NSLOT, COUT, CIN), jnp.float32),
            pltpu.SemaphoreType.DMA((NSLOT,)),
        ],
    )
    return pl.pallas_call(
        _matmul_body,
        grid_spec=grid_spec,
        out_shape=jax.ShapeDtypeStruct((PB, COUT), jnp.float32),
    )(meta, x_padded, W, bias2d)


@functools.partial(
    pl.kernel,
    out_type=jax.ShapeDtypeStruct((B_TOKENS, COUT), jnp.float32),
    mesh=_MESH,
    compiler_params=_SC_PARAMS,
    scratch_types=(
        pltpu.VMEM((2, DCH), jnp.int32),
        pltpu.VMEM((DCH, COUT), jnp.float32),
        pltpu.SemaphoreType.DMA,
    ),
)
def _unpermute(opad_h, pos_h, out_h, pos_v, ob_v, sem):
    w = _worker_id()
    pltpu.sync_copy(pos_h.at[pl.ds(w * 2, 2)], pos_v)
    for c in range(2):
        pltpu.async_copy(opad_h.at[pos_v.at[c]], ob_v, sem).wait()
        pltpu.sync_copy(ob_v, out_h.at[pl.ds(w * TOK_W + c * DCH, DCH)])


def kernel(in_feats, in_coords, W, bias):
    tiles_h, ranks_h, lcounts_h = _route(in_coords)
    x_padded, pos_h, meta = _dispatch(in_feats, tiles_h, ranks_h, lcounts_h)
    out_padded = _grouped_matmul(meta, x_padded, W, bias.reshape(1, COUT))
    return _unpermute(out_padded, pos_h)


# speedup vs baseline: 1.1135x; 1.1135x over previous
import functools

import jax
import jax.numpy as jnp
from jax import lax
from jax.experimental import pallas as pl
from jax.experimental.pallas import tpu as pltpu
from jax.experimental.pallas import tpu_sc as plsc

N_TILES = 64
HGRID = 8
CIN = 768
COUT = 768
B_TOKENS = 8192
A_SCALE = 16.0
A_BIAS = 0.5

R = 128
NBLK = 128
NSUP = NBLK // 2
PB = NBLK * R
CHALF = COUT // 2
DCH = 128

NC = 2
NS = 16
NW = NC * NS
TOK_W = B_TOKENS // NW
NVEC = TOK_W // 16

_MESH = plsc.VectorSubcoreMesh(core_axis_name="c", subcore_axis_name="s",
                               num_cores=NC, num_subcores=NS)
_SC_PARAMS = pltpu.CompilerParams(needs_layout_passes=False)


def _worker_id():
    return lax.axis_index("s") * NC + lax.axis_index("c")


def _floor_i32(v):
    t = v.astype(jnp.int32)
    return jnp.where(t.astype(jnp.float32) > v, t - 1, t)


@functools.partial(
    pl.kernel,
    out_type=(
        jax.ShapeDtypeStruct((B_TOKENS,), jnp.int32),
        jax.ShapeDtypeStruct((B_TOKENS,), jnp.int32),
        jax.ShapeDtypeStruct((NW, N_TILES), jnp.int32),
    ),
    mesh=_MESH,
    compiler_params=_SC_PARAMS,
    scratch_types=(
        pltpu.VMEM((TOK_W, 2), jnp.float32),
        pltpu.VMEM((TOK_W,), jnp.int32),
        pltpu.VMEM((TOK_W,), jnp.int32),
        pltpu.VMEM((N_TILES,), jnp.int32),
    ),
)
def _route(coords, tiles_h, ranks_h, lcounts_h, cc_v, tl_v, rk_v, cnt_v):
    w = _worker_id()
    base = w * TOK_W
    pltpu.sync_copy(coords.at[pl.ds(base, TOK_W)], cc_v)
    for c in range(N_TILES // 16):
        cnt_v[pl.ds(c * 16, 16)] = jnp.zeros((16,), jnp.int32)

    lane = lax.iota(jnp.int32, 16)
    zeros16 = jnp.zeros((16,), jnp.int32)

    def body(k, _):
        sl = pl.ds(k * 16, 16)
        tok = lane + k * 16
        cx = plsc.load_gather(cc_v, [tok, zeros16])
        cy = plsc.load_gather(cc_v, [tok, zeros16 + 1])
        mx = _floor_i32(cx * A_SCALE + A_BIAS) & (HGRID - 1)
        my = _floor_i32(cy * A_SCALE + A_BIAS) & (HGRID - 1)
        tile = mx * HGRID + my
        old = plsc.load_gather(cnt_v, [tile])
        rank = jnp.zeros((16,), jnp.int32)
        total = jnp.zeros((16,), jnp.int32)
        for l in range(16):
            tl = jnp.sum(jnp.where(lane == l, tile, 0))
            eq = tile == tl
            rank = rank + jnp.where(eq & (lane > l), 1, 0)
            total = total + jnp.where(eq, 1, 0)
        tl_v[sl] = tile
        rk_v[sl] = old + rank
        plsc.store_scatter(cnt_v, [tile], old + total)
        return 0

    lax.fori_loop(0, NVEC, body, 0)
    pltpu.sync_copy(tl_v, tiles_h.at[pl.ds(base, TOK_W)])
    pltpu.sync_copy(rk_v, ranks_h.at[pl.ds(base, TOK_W)])
    pltpu.sync_copy(cnt_v, lcounts_h.at[w])


@functools.partial(
    pl.kernel,
    out_type=(
        jax.ShapeDtypeStruct((PB, CIN), jnp.float32),
        jax.ShapeDtypeStruct((NW * 2, DCH), jnp.int32),
        jax.ShapeDtypeStruct((3, NBLK), jnp.int32),
    ),
    mesh=_MESH,
    compiler_params=_SC_PARAMS,
    scratch_types=(
        pltpu.VMEM((NW, N_TILES), jnp.int32),
        pltpu.VMEM((N_TILES,), jnp.int32),
        pltpu.VMEM((TOK_W,), jnp.int32),
        pltpu.VMEM((TOK_W,), jnp.int32),
        pltpu.VMEM((2, DCH), jnp.int32),
        pltpu.VMEM((NBLK,), jnp.int32),
        pltpu.VMEM((3, NBLK), jnp.int32),
        pltpu.VMEM((DCH, CIN), jnp.float32),
        pltpu.SemaphoreType.DMA,
    ),
)
def _dispatch(in_feats, tiles_h, ranks_h, lcounts_h,
              xpad_h, pos_h, meta_h,
              lc_v, base_v, tl_v, rk_v, pos_v, tm_v, meta_v, xb_v, sem):
    w = _worker_id()
    base = w * TOK_W
    pltpu.sync_copy(lcounts_h, lc_v)
    pltpu.sync_copy(tiles_h.at[pl.ds(base, TOK_W)], tl_v)
    pltpu.sync_copy(ranks_h.at[pl.ds(base, TOK_W)], rk_v)

    lane = lax.iota(jnp.int32, 16)
    for c in range(NBLK // 16):
        tm_v[pl.ds(c * 16, 16)] = jnp.zeros((16,), jnp.int32)
    used = jnp.int32(0)
    carry = jnp.int32(0)
    for c in range(N_TILES // 16):
        sl = pl.ds(c * 16, 16)

        def red(wp, acc):
            tot, pre = acc
            v = lc_v[wp, sl]
            tot = tot + v
            pre = pre + jnp.where(wp < w, v, 0)
            return (tot, pre)

        tot, pre = lax.fori_loop(
            0, NW, red, (jnp.zeros((16,), jnp.int32), jnp.zeros((16,), jnp.int32)))
        nb = (tot + (R - 1)) // R
        bstart = jnp.cumsum(nb) - nb + carry
        carry = carry + jnp.sum(nb)
        base_v[sl] = bstart * R + pre
        tvec = lane + c * 16
        plsc.store_scatter(tm_v, [jnp.minimum(bstart, NBLK - 1)],
                           tvec, mask=nb > 0)
        used = used + jnp.sum(nb)

    for k in range(NVEC):
        sl = pl.ds((k % (NVEC // 2)) * 16, 16)
        t = tl_v[pl.ds(k * 16, 16)]
        p = plsc.load_gather(base_v, [t]) + rk_v[pl.ds(k * 16, 16)]
        pos_v[k // (NVEC // 2), sl] = p

    for c in range(2):
        pltpu.sync_copy(in_feats.at[pl.ds(base + c * DCH, DCH)], xb_v)
        pltpu.async_copy(xb_v, xpad_h.at[pos_v.at[c]], sem).wait()
        pltpu.sync_copy(pos_v.at[c], pos_h.at[w * 2 + c])

    @pl.when(w == 0)
    def _():
        cmax = jnp.int32(0)
        for c in range(NBLK // 16):
            sl = pl.ds(c * 16, 16)
            v = jnp.maximum(plsc.cummax(tm_v[sl]), cmax)
            meta_v[0, sl] = v
            tm_v[sl] = v
            cmax = jnp.max(v)
        for c in range(NSUP // 16):
            sl = pl.ds(c * 16, 16)
            blk = lane + c * 16
            meta_v[1, sl] = jnp.where(2 * blk < used, blk, NSUP - 1)
        fcarry = jnp.int32(0)
        for c in range(NBLK // 16):
            sl = pl.ds(c * 16, 16)
            blk = lane + c * 16
            cur = tm_v[sl]
            prev = plsc.load_gather(tm_v, [jnp.maximum(blk - 1, 0)])
            diff = jnp.where((cur != prev) & (blk > 0), 1, 0)
            fid = jnp.cumsum(diff) + fcarry
            meta_v[2, sl] = fid
            fcarry = jnp.max(fid)
        pltpu.sync_copy(meta_v, meta_h)


NSLOT = 6


def _matmul_body(meta_ref, x_ref, w_hbm, b_ref, o_ref, wbuf, sems):
    i = pl.program_id(0)
    fa = meta_ref[2, 2 * i]
    fb = meta_ref[2, 2 * i + 1]
    sa = lax.rem(fa, NSLOT)
    sb = lax.rem(fb, NSLOT)

    def start_fetch(j):
        fj = meta_ref[2, j]
        s = lax.rem(fj, NSLOT)
        pltpu.make_async_copy(
            w_hbm.at[meta_ref[0, j]], wbuf.at[s], sems.at[s]).start()

    @pl.when(i == 0)
    def _():
        start_fetch(0)
        for j in range(1, 6):
            @pl.when(meta_ref[2, j] > meta_ref[2, j - 1])
            def _(j=j):
                start_fetch(j)

    @pl.when((i > 0) & (i + 2 < NSUP))
    def _():
        for jj in (2 * i + 4, 2 * i + 5):
            @pl.when(meta_ref[2, jj] > meta_ref[2, jj - 1])
            def _(jj=jj):
                start_fetch(jj)

    prev_f = meta_ref[2, jnp.maximum(2 * i - 1, 0)]

    @pl.when((i == 0) | (fa > prev_f))
    def _():
        pltpu.make_async_copy(
            w_hbm.at[meta_ref[0, 2 * i]], wbuf.at[sa], sems.at[sa]).wait()

    @pl.when(fb > fa)
    def _():
        pltpu.make_async_copy(
            w_hbm.at[meta_ref[0, 2 * i + 1]], wbuf.at[sb], sems.at[sb]).wait()

    x = x_ref[...]
    b = b_ref[...]
    dn = (((1,), (1,)), ((), ()))

    @pl.when(fa == fb)
    def _():
        acc = lax.dot_general(x, wbuf[sa], dn,
                              preferred_element_type=jnp.float32) + b
        o_ref[...] = jnp.where(acc >= 0, acc, 0.2 * acc)

    @pl.when(fa != fb)
    def _():
        a0 = lax.dot_general(x[:R], wbuf[sa], dn,
                             preferred_element_type=jnp.float32) + b
        o_ref[:R, :] = jnp.where(a0 >= 0, a0, 0.2 * a0)
        a1 = lax.dot_general(x[R:], wbuf[sb], dn,
                             preferred_element_type=jnp.float32) + b
        o_ref[R:, :] = jnp.where(a1 >= 0, a1, 0.2 * a1)


def _grouped_matmul(meta, x_padded, W, bias2d):
    grid_spec = pltpu.PrefetchScalarGridSpec(
        num_scalar_prefetch=1,
        grid=(NSUP,),
        in_specs=[
            pl.BlockSpec((2 * R, CIN), lambda i, m: (m[1, i], 0)),
            pl.BlockSpec(memory_space=pl.ANY),
            pl.BlockSpec((1, COUT), lambda i, m: (0, 0)),
        ],
        out_specs=pl.BlockSpec((2 * R, COUT), lambda i, m: (m[1, i], 0)),
        scratch_shapes=[
            pltpu.VMEM((NSLOT, COUT, CIN), jnp.float32),
            pltpu.SemaphoreType.DMA((NSLOT,)),
        ],
    )
    return pl.pallas_call(
        _matmul_body,
        grid_spec=grid_spec,
        out_shape=jax.ShapeDtypeStruct((PB, COUT), jnp.float32),
    )(meta, x_padded, W, bias2d)


@functools.partial(
    pl.kernel,
    out_type=jax.ShapeDtypeStruct((B_TOKENS, COUT), jnp.float32),
    mesh=_MESH,
    compiler_params=_SC_PARAMS,
    scratch_types=(
        pltpu.VMEM((2, DCH), jnp.int32),
        pltpu.VMEM((DCH, COUT), jnp.float32),
        pltpu.SemaphoreType.DMA,
    ),
)
def _unpermute(opad_h, pos_h, out_h, pos_v, ob_v, sem):
    w = _worker_id()
    pltpu.sync_copy(pos_h.at[pl.ds(w * 2, 2)], pos_v)
    for c in range(2):
        pltpu.async_copy(opad_h.at[pos_v.at[c]], ob_v, sem).wait()
        pltpu.sync_copy(ob_v, out_h.at[pl.ds(w * TOK_W + c * DCH, DCH)])


def kernel(in_feats, in_coords, W, bias):
    tiles_h, ranks_h, lcounts_h = _route(in_coords)
    x_padded, pos_h, meta = _dispatch(in_feats, tiles_h, ranks_h, lcounts_h)
    out_padded = _grouped_matmul(meta, x_padded, W, bias.reshape(1, COUT))
    return _unpermute(out_padded, pos_h)


# speedup vs baseline: 1.1163x; 1.0025x over previous
import functools

import jax
import jax.numpy as jnp
from jax import lax
from jax.experimental import pallas as pl
from jax.experimental.pallas import tpu as pltpu
from jax.experimental.pallas import tpu_sc as plsc

N_TILES = 64
HGRID = 8
CIN = 768
COUT = 768
B_TOKENS = 8192
A_SCALE = 16.0
A_BIAS = 0.5

R = 128
NBLK = 128
NSUP = NBLK // 2
PB = NBLK * R
DCH = 128

NC = 2
NS = 16
NW = NC * NS
TOK_W = B_TOKENS // NW
NVEC = TOK_W // 16

_MESH = plsc.VectorSubcoreMesh(core_axis_name="c", subcore_axis_name="s",
                               num_cores=NC, num_subcores=NS)
_SC_PARAMS = pltpu.CompilerParams(needs_layout_passes=False)


def _worker_id():
    return lax.axis_index("s") * NC + lax.axis_index("c")


def _floor_i32(v):
    t = v.astype(jnp.int32)
    return jnp.where(t.astype(jnp.float32) > v, t - 1, t)


@functools.partial(
    pl.kernel,
    out_type=(
        jax.ShapeDtypeStruct((B_TOKENS,), jnp.int32),
        jax.ShapeDtypeStruct((B_TOKENS,), jnp.int32),
        jax.ShapeDtypeStruct((NW, N_TILES), jnp.int32),
    ),
    mesh=_MESH,
    compiler_params=_SC_PARAMS,
    scratch_types=(
        pltpu.VMEM((TOK_W, 2), jnp.float32),
        pltpu.VMEM((TOK_W,), jnp.int32),
        pltpu.VMEM((TOK_W,), jnp.int32),
        pltpu.VMEM((N_TILES,), jnp.int32),
    ),
)
def _route(coords, tiles_h, ranks_h, lcounts_h, cc_v, tl_v, rk_v, cnt_v):
    w = _worker_id()
    base = w * TOK_W
    pltpu.sync_copy(coords.at[pl.ds(base, TOK_W)], cc_v)
    for c in range(N_TILES // 16):
        cnt_v[pl.ds(c * 16, 16)] = jnp.zeros((16,), jnp.int32)

    lane = lax.iota(jnp.int32, 16)
    zeros16 = jnp.zeros((16,), jnp.int32)

    def body(k, _):
        sl = pl.ds(k * 16, 16)
        tok = lane + k * 16
        cx = plsc.load_gather(cc_v, [tok, zeros16])
        cy = plsc.load_gather(cc_v, [tok, zeros16 + 1])
        mx = _floor_i32(cx * A_SCALE + A_BIAS) & (HGRID - 1)
        my = _floor_i32(cy * A_SCALE + A_BIAS) & (HGRID - 1)
        tile = mx * HGRID + my
        old = plsc.load_gather(cnt_v, [tile])
        rank = jnp.zeros((16,), jnp.int32)
        total = jnp.zeros((16,), jnp.int32)
        for l in range(16):
            tl = jnp.sum(jnp.where(lane == l, tile, 0))
            eq = tile == tl
            rank = rank + jnp.where(eq & (lane > l), 1, 0)
            total = total + jnp.where(eq, 1, 0)
        tl_v[sl] = tile
        rk_v[sl] = old + rank
        plsc.store_scatter(cnt_v, [tile], old + total)
        return 0

    lax.fori_loop(0, NVEC, body, 0)
    pltpu.sync_copy(tl_v, tiles_h.at[pl.ds(base, TOK_W)])
    pltpu.sync_copy(rk_v, ranks_h.at[pl.ds(base, TOK_W)])
    pltpu.sync_copy(cnt_v, lcounts_h.at[w])


@functools.partial(
    pl.kernel,
    out_type=(
        jax.ShapeDtypeStruct((PB, CIN), jnp.float32),
        jax.ShapeDtypeStruct((NW * 2, DCH), jnp.int32),
        jax.ShapeDtypeStruct((3, NBLK), jnp.int32),
    ),
    mesh=_MESH,
    compiler_params=_SC_PARAMS,
    scratch_types=(
        pltpu.VMEM((NW, N_TILES), jnp.int32),
        pltpu.VMEM((N_TILES,), jnp.int32),
        pltpu.VMEM((TOK_W,), jnp.int32),
        pltpu.VMEM((TOK_W,), jnp.int32),
        pltpu.VMEM((2, DCH), jnp.int32),
        pltpu.VMEM((NBLK,), jnp.int32),
        pltpu.VMEM((3, NBLK), jnp.int32),
        pltpu.VMEM((DCH, CIN), jnp.float32),
        pltpu.SemaphoreType.DMA,
    ),
)
def _dispatch(in_feats, tiles_h, ranks_h, lcounts_h,
              xpad_h, pos_h, meta_h,
              lc_v, base_v, tl_v, rk_v, pos_v, tm_v, meta_v, xb_v, sem):
    w = _worker_id()
    base = w * TOK_W
    pltpu.sync_copy(lcounts_h, lc_v)
    pltpu.sync_copy(tiles_h.at[pl.ds(base, TOK_W)], tl_v)
    pltpu.sync_copy(ranks_h.at[pl.ds(base, TOK_W)], rk_v)

    lane = lax.iota(jnp.int32, 16)
    for c in range(NBLK // 16):
        tm_v[pl.ds(c * 16, 16)] = jnp.zeros((16,), jnp.int32)
    used = jnp.int32(0)
    carry = jnp.int32(0)
    for c in range(N_TILES // 16):
        sl = pl.ds(c * 16, 16)

        def red(wp, acc):
            tot, pre = acc
            v = lc_v[wp, sl]
            tot = tot + v
            pre = pre + jnp.where(wp < w, v, 0)
            return (tot, pre)

        tot, pre = lax.fori_loop(
            0, NW, red, (jnp.zeros((16,), jnp.int32), jnp.zeros((16,), jnp.int32)))
        nb = (tot + (R - 1)) // R
        bstart = jnp.cumsum(nb) - nb + carry
        carry = carry + jnp.sum(nb)
        base_v[sl] = bstart * R + pre
        tvec = lane + c * 16
        plsc.store_scatter(tm_v, [jnp.minimum(bstart, NBLK - 1)],
                           tvec, mask=nb > 0)
        used = used + jnp.sum(nb)

    for k in range(NVEC):
        sl = pl.ds((k % (NVEC // 2)) * 16, 16)
        t = tl_v[pl.ds(k * 16, 16)]
        p = plsc.load_gather(base_v, [t]) + rk_v[pl.ds(k * 16, 16)]
        pos_v[k // (NVEC // 2), sl] = p

    for c in range(2):
        pltpu.sync_copy(in_feats.at[pl.ds(base + c * DCH, DCH)], xb_v)
        pltpu.async_copy(xb_v, xpad_h.at[pos_v.at[c]], sem).wait()
        pltpu.sync_copy(pos_v.at[c], pos_h.at[w * 2 + c])

    @pl.when(w == 0)
    def _():
        cmax = jnp.int32(0)
        for c in range(NBLK // 16):
            sl = pl.ds(c * 16, 16)
            v = jnp.maximum(plsc.cummax(tm_v[sl]), cmax)
            meta_v[0, sl] = v
            tm_v[sl] = v
            cmax = jnp.max(v)
        for c in range(NSUP // 16):
            sl = pl.ds(c * 16, 16)
            blk = lane + c * 16
            meta_v[1, sl] = jnp.where(2 * blk < used, blk, NSUP - 1)
        fcarry = jnp.int32(0)
        for c in range(NBLK // 16):
            sl = pl.ds(c * 16, 16)
            blk = lane + c * 16
            cur = tm_v[sl]
            prev = plsc.load_gather(tm_v, [jnp.maximum(blk - 1, 0)])
            diff = jnp.where((cur != prev) & (blk > 0), 1, 0)
            fid = jnp.cumsum(diff) + fcarry
            meta_v[2, sl] = fid
            fcarry = jnp.max(fid)
        pltpu.sync_copy(meta_v, meta_h)


NSLOT = 6


def _matmul_body(meta_ref, x_ref, w_hbm, b_ref, o_ref, wbuf, sems):
    i = pl.program_id(0)
    fa = meta_ref[2, 2 * i]
    fb = meta_ref[2, 2 * i + 1]
    sa = lax.rem(fa, NSLOT)
    sb = lax.rem(fb, NSLOT)

    def start_fetch(j):
        fj = meta_ref[2, j]
        s = lax.rem(fj, NSLOT)
        pltpu.make_async_copy(
            w_hbm.at[meta_ref[0, j]], wbuf.at[s], sems.at[s]).start()

    @pl.when(i == 0)
    def _():
        start_fetch(0)
        for j in range(1, 6):
            @pl.when(meta_ref[2, j] > meta_ref[2, j - 1])
            def _(j=j):
                start_fetch(j)

    @pl.when((i > 0) & (i + 2 < NSUP))
    def _():
        for jj in (2 * i + 4, 2 * i + 5):
            @pl.when(meta_ref[2, jj] > meta_ref[2, jj - 1])
            def _(jj=jj):
                start_fetch(jj)

    prev_f = meta_ref[2, jnp.maximum(2 * i - 1, 0)]

    @pl.when((i == 0) | (fa > prev_f))
    def _():
        pltpu.make_async_copy(
            w_hbm.at[meta_ref[0, 2 * i]], wbuf.at[sa], sems.at[sa]).wait()

    @pl.when(fb > fa)
    def _():
        pltpu.make_async_copy(
            w_hbm.at[meta_ref[0, 2 * i + 1]], wbuf.at[sb], sems.at[sb]).wait()

    x = x_ref[...]
    b = b_ref[...]
    dn = (((1,), (1,)), ((), ()))

    @pl.when(fa == fb)
    def _():
        acc = lax.dot_general(x, wbuf[sa], dn,
                              preferred_element_type=jnp.float32) + b
        o_ref[...] = jnp.where(acc >= 0, acc, 0.2 * acc)

    @pl.when(fa != fb)
    def _():
        a0 = lax.dot_general(x[:R], wbuf[sa], dn,
                             preferred_element_type=jnp.float32) + b
        o_ref[:R, :] = jnp.where(a0 >= 0, a0, 0.2 * a0)
        a1 = lax.dot_general(x[R:], wbuf[sb], dn,
                             preferred_element_type=jnp.float32) + b
        o_ref[R:, :] = jnp.where(a1 >= 0, a1, 0.2 * a1)


def _grouped_matmul(meta, x_padded, W, bias2d):
    grid_spec = pltpu.PrefetchScalarGridSpec(
        num_scalar_prefetch=1,
        grid=(NSUP,),
        in_specs=[
            pl.BlockSpec((2 * R, CIN), lambda i, m: (m[1, i], 0)),
            pl.BlockSpec(memory_space=pl.ANY),
            pl.BlockSpec((1, COUT), lambda i, m: (0, 0)),
        ],
        out_specs=pl.BlockSpec((2 * R, COUT), lambda i, m: (m[1, i], 0)),
        scratch_shapes=[
            pltpu.VMEM((NSLOT, COUT, CIN), jnp.float32),
            pltpu.SemaphoreType.DMA((NSLOT,)),
        ],
    )
    return pl.pallas_call(
        _matmul_body,
        grid_spec=grid_spec,
        out_shape=jax.ShapeDtypeStruct((PB, COUT), jnp.float32),
    )(meta, x_padded, W, bias2d)


@functools.partial(
    pl.kernel,
    out_type=jax.ShapeDtypeStruct((B_TOKENS, COUT), jnp.float32),
    mesh=_MESH,
    compiler_params=_SC_PARAMS,
    scratch_types=(
        pltpu.VMEM((2, DCH), jnp.int32),
        pltpu.VMEM((DCH, COUT), jnp.float32),
        pltpu.SemaphoreType.DMA,
    ),
)
def _unpermute(opad_h, pos_h, out_h, pos_v, ob_v, sem):
    w = _worker_id()
    pltpu.sync_copy(pos_h.at[pl.ds(w * 2, 2)], pos_v)
    for c in range(2):
        pltpu.async_copy(opad_h.at[pos_v.at[c]], ob_v, sem).wait()
        pltpu.sync_copy(ob_v, out_h.at[pl.ds(w * TOK_W + c * DCH, DCH)])


def kernel(in_feats, in_coords, W, bias):
    tiles_h, ranks_h, lcounts_h = _route(in_coords)
    x_padded, pos_h, meta = _dispatch(in_feats, tiles_h, ranks_h, lcounts_h)
    out_padded = _grouped_matmul(meta, x_padded, W, bias.reshape(1, COUT))
    return _unpermute(out_padded, pos_h)
